# Initial kernel scaffold; baseline (speedup 1.0000x reference)
import jax
import jax.numpy as jnp
from jax.experimental import pallas as pl


def kernel(values, moment_idx, carry):
    raise NotImplementedError("write your pallas kernel here")


# speedup vs baseline: 65.7263x; 65.7263x over previous
import functools

import jax
import jax.numpy as jnp
from jax import lax
from jax.experimental import pallas as pl
from jax.experimental.pallas import tpu as pltpu
from jax.experimental.pallas import tpu_sc as plsc

B = 8
N = 50000
G = 1600000
NC = 2
NS = 16
NCHUNK = 16
CH = G // NCHUNK
R = 5
CHR = CH // R
S = 2000
NSUBR = CHR // S
VPS = S // 16
WR = (8 * CHR) // NS
PW = 400
NP = WR // PW


def _body(tbl_hbm, idx_hbm, carry_hbm, out_hbm,
          p0, p1, idxv, accw, wa, wb, wc, shared):
    c = lax.axis_index("c")
    s = lax.axis_index("s")
    pg = s // 8
    q = s % 8
    gbase = (c * 8 + q) * CH

    pltpu.sync_copy(tbl_hbm.at[2 * pg], p0)
    pltpu.sync_copy(tbl_hbm.at[2 * pg + 1], p1)

    def dot4(ii, jj):
        wi0 = plsc.load_gather(p0, [ii])
        wj0 = plsc.load_gather(p0, [jj])
        wi1 = plsc.load_gather(p1, [ii])
        wj1 = plsc.load_gather(p1, [jj])
        ai, bi = plsc.unpack(plsc.bitcast(wi0, jnp.bfloat16),
                             format=plsc.PackFormat.INTERLEAVED)
        aj, bj = plsc.unpack(plsc.bitcast(wj0, jnp.bfloat16),
                             format=plsc.PackFormat.INTERLEAVED)
        ci, di = plsc.unpack(plsc.bitcast(wi1, jnp.bfloat16),
                             format=plsc.PackFormat.INTERLEAVED)
        cj, dj = plsc.unpack(plsc.bitcast(wj1, jnp.bfloat16),
                             format=plsc.PackFormat.INTERLEAVED)
        return ai * aj + bi * bj + ci * cj + di * dj

    def unpack_idx(w):
        ii = lax.convert_element_type(w & jnp.uint32(0xFFFF), jnp.int32)
        jj = lax.convert_element_type(
            jnp.right_shift(w, jnp.uint32(16)), jnp.int32)
        return ii, jj

    def round_body(r, carry0):
        rbase = gbase + r * CHR

        def sub_body(sub, carry_):
            pltpu.sync_copy(
                idx_hbm.at[pl.ds(pl.multiple_of(rbase + sub * S, 8), S)],
                idxv)

            def vec_body(v, carry__):
                accw[pl.ds(v * 16, 16)] = dot4(
                    *unpack_idx(idxv[pl.ds(v * 16, 16)]))
                return carry__

            lax.fori_loop(0, VPS, vec_body, 0)
            pltpu.sync_copy(
                accw,
                shared.at[pl.ds(
                    pl.multiple_of(pg * 8 * CHR + q * CHR + sub * S, 8),
                    S)])
            return carry_

        lax.fori_loop(0, NSUBR, sub_body, 0)
        plsc.subcore_barrier()

        def wo_body(p, carry_):
            off = s * WR + p * PW
            goff = pl.multiple_of(
                (c * 8 + off // CHR) * CH + r * CHR + off % CHR, 8)
            pltpu.sync_copy(
                shared.at[pl.ds(pl.multiple_of(off, 8), PW)], wa)
            pltpu.sync_copy(
                shared.at[pl.ds(pl.multiple_of(8 * CHR + off, 8), PW)], wb)
            pltpu.sync_copy(carry_hbm.at[pl.ds(goff, PW)], wc)

            def add_body(t, carry__):
                sl = pl.ds(t * 16, 16)
                wc[sl] = wa[sl] + wb[sl] + wc[sl]
                return carry__

            lax.fori_loop(0, PW // 16, add_body, 0)
            pltpu.sync_copy(wc, out_hbm.at[pl.ds(goff, PW)])
            return carry_

        lax.fori_loop(0, NP, wo_body, 0)
        plsc.subcore_barrier()
        return carry0

    lax.fori_loop(0, R, round_body, 0)


def kernel(values, moment_idx, carry):
    vb = values.astype(jnp.bfloat16)
    u = lax.bitcast_convert_type(vb, jnp.uint16).astype(jnp.uint32)
    words = u[0::2] | (u[1::2] << jnp.uint32(16))
    tbl = lax.bitcast_convert_type(words, jnp.float32)

    mi = moment_idx.astype(jnp.uint32)
    idxp = mi[:, 0] | (mi[:, 1] << jnp.uint32(16))

    run = functools.partial(
        pl.kernel,
        out_type=jax.ShapeDtypeStruct((G,), jnp.float32),
        mesh=plsc.VectorSubcoreMesh(core_axis_name="c", subcore_axis_name="s"),
        compiler_params=pltpu.CompilerParams(needs_layout_passes=False),
        scratch_types=[
            pltpu.VMEM((N,), jnp.float32),
            pltpu.VMEM((N,), jnp.float32),
            pltpu.VMEM((S,), jnp.uint32),
            pltpu.VMEM((S,), jnp.float32),
            pltpu.VMEM((PW,), jnp.float32),
            pltpu.VMEM((PW,), jnp.float32),
            pltpu.VMEM((PW,), jnp.float32),
            pltpu.VMEM_SHARED((2 * 8 * CHR,), jnp.float32),
        ],
    )(_body)
    return run(tbl, idxp, carry)


# speedup vs baseline: 76.4575x; 1.1633x over previous
import functools

import jax
import jax.numpy as jnp
from jax import lax
from jax.experimental import pallas as pl
from jax.experimental.pallas import tpu as pltpu
from jax.experimental.pallas import tpu_sc as plsc

B = 8
N = 50000
G = 1600000
NC = 2
NS = 16
NCHUNK = 16
CH = G // NCHUNK
R = 5
CHR = CH // R
S = 2000
NSUBR = CHR // S
VPS = S // 16
WR = (8 * CHR) // NS
PW = 400
NP = WR // PW


def _body(tbl_hbm, idx_hbm, carry_hbm, out_hbm,
          p0, p1, idxv, accw, wa, wb, wc, shared):
    c = lax.axis_index("c")
    s = lax.axis_index("s")
    pg = s // 8
    q = s % 8
    gbase = (c * 8 + q) * CH

    pltpu.sync_copy(tbl_hbm.at[2 * pg], p0)
    pltpu.sync_copy(tbl_hbm.at[2 * pg + 1], p1)

    def dot4(ii, jj):
        wi0 = plsc.load_gather(p0, [ii])
        wj0 = plsc.load_gather(p0, [jj])
        wi1 = plsc.load_gather(p1, [ii])
        wj1 = plsc.load_gather(p1, [jj])
        ai, bi = plsc.unpack(plsc.bitcast(wi0, jnp.bfloat16),
                             format=plsc.PackFormat.INTERLEAVED)
        aj, bj = plsc.unpack(plsc.bitcast(wj0, jnp.bfloat16),
                             format=plsc.PackFormat.INTERLEAVED)
        ci, di = plsc.unpack(plsc.bitcast(wi1, jnp.bfloat16),
                             format=plsc.PackFormat.INTERLEAVED)
        cj, dj = plsc.unpack(plsc.bitcast(wj1, jnp.bfloat16),
                             format=plsc.PackFormat.INTERLEAVED)
        return ai * aj + bi * bj + ci * cj + di * dj

    def unpack_idx(w):
        ii = lax.convert_element_type(w & jnp.uint32(0xFFFF), jnp.int32)
        jj = lax.convert_element_type(
            jnp.right_shift(w, jnp.uint32(16)), jnp.int32)
        return ii, jj

    def round_body(r, carry0):
        rbase = gbase + r * CHR

        def sub_body(sub, carry_):
            pltpu.sync_copy(
                idx_hbm.at[pl.ds(pl.multiple_of(rbase + sub * S, 8), S)],
                idxv)

            @plsc.parallel_loop(0, VPS, unroll=8)
            def vec_body(v):
                accw[pl.ds(v * 16, 16)] = dot4(
                    *unpack_idx(idxv[pl.ds(v * 16, 16)]))
            pltpu.sync_copy(
                accw,
                shared.at[pl.ds(
                    pl.multiple_of(pg * 8 * CHR + q * CHR + sub * S, 8),
                    S)])
            return carry_

        lax.fori_loop(0, NSUBR, sub_body, 0)
        plsc.subcore_barrier()

        def wo_body(p, carry_):
            off = s * WR + p * PW
            goff = pl.multiple_of(
                (c * 8 + off // CHR) * CH + r * CHR + off % CHR, 8)
            pltpu.sync_copy(
                shared.at[pl.ds(pl.multiple_of(off, 8), PW)], wa)
            pltpu.sync_copy(
                shared.at[pl.ds(pl.multiple_of(8 * CHR + off, 8), PW)], wb)
            pltpu.sync_copy(carry_hbm.at[pl.ds(goff, PW)], wc)

            @plsc.parallel_loop(0, PW // 16, unroll=5)
            def add_body(t):
                sl = pl.ds(t * 16, 16)
                wc[sl] = wa[sl] + wb[sl] + wc[sl]
            pltpu.sync_copy(wc, out_hbm.at[pl.ds(goff, PW)])
            return carry_

        lax.fori_loop(0, NP, wo_body, 0)
        plsc.subcore_barrier()
        return carry0

    lax.fori_loop(0, R, round_body, 0)


def kernel(values, moment_idx, carry):
    vb = values.astype(jnp.bfloat16)
    u = lax.bitcast_convert_type(vb, jnp.uint16).astype(jnp.uint32)
    words = u[0::2] | (u[1::2] << jnp.uint32(16))
    tbl = lax.bitcast_convert_type(words, jnp.float32)

    mi = moment_idx.astype(jnp.uint32)
    idxp = mi[:, 0] | (mi[:, 1] << jnp.uint32(16))

    run = functools.partial(
        pl.kernel,
        out_type=jax.ShapeDtypeStruct((G,), jnp.float32),
        mesh=plsc.VectorSubcoreMesh(core_axis_name="c", subcore_axis_name="s"),
        compiler_params=pltpu.CompilerParams(needs_layout_passes=False),
        scratch_types=[
            pltpu.VMEM((N,), jnp.float32),
            pltpu.VMEM((N,), jnp.float32),
            pltpu.VMEM((S,), jnp.uint32),
            pltpu.VMEM((S,), jnp.float32),
            pltpu.VMEM((PW,), jnp.float32),
            pltpu.VMEM((PW,), jnp.float32),
            pltpu.VMEM((PW,), jnp.float32),
            pltpu.VMEM_SHARED((2 * 8 * CHR,), jnp.float32),
        ],
    )(_body)
    return run(tbl, idxp, carry)


# speedup vs baseline: 100.3807x; 1.3129x over previous
import functools

import jax
import jax.numpy as jnp
from jax import lax
from jax.experimental import pallas as pl
from jax.experimental.pallas import tpu as pltpu
from jax.experimental.pallas import tpu_sc as plsc

B = 8
N = 50000
G = 1600000
NC = 2
NS = 16
NCHUNK = 16
CH = G // NCHUNK
R = 5
CHR = CH // R
S = 800
NSUBR = CHR // S
VPS = S // 16
WR = (8 * CHR) // NS
PW = 2000
NP = WR // PW


def _body(tbl_hbm, idx_hbm, carry_hbm, out_hbm,
          p0, p1, idxa, idxb, accw, wa, wb, wc, shared, sema, semb):
    c = lax.axis_index("c")
    s = lax.axis_index("s")
    pg = s // 8
    q = s % 8
    gbase = (c * 8 + q) * CH

    pltpu.sync_copy(tbl_hbm.at[2 * pg], p0)
    pltpu.sync_copy(tbl_hbm.at[2 * pg + 1], p1)

    def dot4(ii, jj):
        wi0 = plsc.load_gather(p0, [ii])
        wj0 = plsc.load_gather(p0, [jj])
        wi1 = plsc.load_gather(p1, [ii])
        wj1 = plsc.load_gather(p1, [jj])
        ai, bi = plsc.unpack(plsc.bitcast(wi0, jnp.bfloat16),
                             format=plsc.PackFormat.INTERLEAVED)
        aj, bj = plsc.unpack(plsc.bitcast(wj0, jnp.bfloat16),
                             format=plsc.PackFormat.INTERLEAVED)
        ci, di = plsc.unpack(plsc.bitcast(wi1, jnp.bfloat16),
                             format=plsc.PackFormat.INTERLEAVED)
        cj, dj = plsc.unpack(plsc.bitcast(wj1, jnp.bfloat16),
                             format=plsc.PackFormat.INTERLEAVED)
        return ai * aj + bi * bj + ci * cj + di * dj

    def unpack_idx(w):
        ii = lax.convert_element_type(w & jnp.uint32(0xFFFF), jnp.int32)
        jj = lax.convert_element_type(
            jnp.right_shift(w, jnp.uint32(16)), jnp.int32)
        return ii, jj

    def round_body(r, carry0):
        rbase = gbase + r * CHR

        def idx_src(sub):
            return idx_hbm.at[pl.ds(pl.multiple_of(rbase + sub * S, 8), S)]

        def compute_sub(sub, idxv):
            @plsc.parallel_loop(0, VPS, unroll=8)
            def vec_body(v):
                accw[pl.ds(v * 16, 16)] = dot4(
                    *unpack_idx(idxv[pl.ds(v * 16, 16)]))
            pltpu.sync_copy(
                accw,
                shared.at[pl.ds(
                    pl.multiple_of(pg * 8 * CHR + q * CHR + sub * S, 8),
                    S)])

        pltpu.async_copy(idx_src(0), idxa, sema)

        def pair_body(k, carry_):
            sub0 = 2 * k
            sub1 = 2 * k + 1
            pltpu.make_async_copy(idx_src(sub0), idxa, sema).wait()
            pltpu.async_copy(idx_src(sub1), idxb, semb)
            compute_sub(sub0, idxa)
            pltpu.make_async_copy(idx_src(sub1), idxb, semb).wait()
            pltpu.async_copy(idx_src(sub1 + 1), idxa, sema)
            compute_sub(sub1, idxb)
            return carry_

        lax.fori_loop(0, NSUBR // 2, pair_body, 0)
        pltpu.make_async_copy(idx_src(NSUBR - 1), idxa, sema).wait()
        compute_sub(NSUBR - 1, idxa)
        plsc.subcore_barrier()

        def wo_body(p, carry_):
            off = s * WR + p * PW
            goff = pl.multiple_of(
                (c * 8 + off // CHR) * CH + r * CHR + off % CHR, 8)
            pltpu.sync_copy(
                shared.at[pl.ds(pl.multiple_of(off, 8), PW)], wa)
            pltpu.sync_copy(
                shared.at[pl.ds(pl.multiple_of(8 * CHR + off, 8), PW)], wb)
            pltpu.sync_copy(carry_hbm.at[pl.ds(goff, PW)], wc)

            @plsc.parallel_loop(0, PW // 16, unroll=5)
            def add_body(t):
                sl = pl.ds(t * 16, 16)
                wc[sl] = wa[sl] + wb[sl] + wc[sl]
            pltpu.sync_copy(wc, out_hbm.at[pl.ds(goff, PW)])
            return carry_

        lax.fori_loop(0, NP, wo_body, 0)
        plsc.subcore_barrier()
        return carry0

    lax.fori_loop(0, R, round_body, 0)


def kernel(values, moment_idx, carry):
    vb = values.astype(jnp.bfloat16)
    u = lax.bitcast_convert_type(vb, jnp.uint16).astype(jnp.uint32)
    words = u[0::2] | (u[1::2] << jnp.uint32(16))
    tbl = lax.bitcast_convert_type(words, jnp.float32)

    mi = moment_idx.astype(jnp.uint32)
    idxp = mi[:, 0] | (mi[:, 1] << jnp.uint32(16))

    run = functools.partial(
        pl.kernel,
        out_type=jax.ShapeDtypeStruct((G,), jnp.float32),
        mesh=plsc.VectorSubcoreMesh(core_axis_name="c", subcore_axis_name="s"),
        compiler_params=pltpu.CompilerParams(needs_layout_passes=False),
        scratch_types=[
            pltpu.VMEM((N,), jnp.float32),
            pltpu.VMEM((N,), jnp.float32),
            pltpu.VMEM((S,), jnp.uint32),
            pltpu.VMEM((S,), jnp.uint32),
            pltpu.VMEM((S,), jnp.float32),
            pltpu.VMEM((PW,), jnp.float32),
            pltpu.VMEM((PW,), jnp.float32),
            pltpu.VMEM((PW,), jnp.float32),
            pltpu.VMEM_SHARED((2 * 8 * CHR,), jnp.float32),
            pltpu.SemaphoreType.DMA,
            pltpu.SemaphoreType.DMA,
        ],
    )(_body)
    return run(tbl, idxp, carry)


# speedup vs baseline: 116.6937x; 1.1625x over previous
import functools

import jax
import jax.numpy as jnp
from jax import lax
from jax.experimental import pallas as pl
from jax.experimental.pallas import tpu as pltpu
from jax.experimental.pallas import tpu_sc as plsc

B = 8
N = 50000
G = 1600000
NC = 2
NS = 16
NCHUNK = 16
CH = G // NCHUNK
R = 5
CHR = CH // R
S = 2000
NSUBR = CHR // S
VPS = S // 16
WR = (8 * CHR) // NS
PW = 2000
NP = WR // PW


def _body(tbl_hbm, idx_hbm, carry_hbm, out_hbm,
          p0, p1, idxa, idxb, accw, wc, shared, sema, semb):
    c = lax.axis_index("c")
    s = lax.axis_index("s")
    pg = s // 8
    q = s % 8
    gbase = (c * 8 + q) * CH

    pltpu.sync_copy(tbl_hbm.at[2 * pg], p0)
    pltpu.sync_copy(tbl_hbm.at[2 * pg + 1], p1)

    def dot4(ii, jj):
        wi0 = plsc.load_gather(p0, [ii])
        wj0 = plsc.load_gather(p0, [jj])
        wi1 = plsc.load_gather(p1, [ii])
        wj1 = plsc.load_gather(p1, [jj])
        ai, bi = plsc.unpack(plsc.bitcast(wi0, jnp.bfloat16),
                             format=plsc.PackFormat.INTERLEAVED)
        aj, bj = plsc.unpack(plsc.bitcast(wj0, jnp.bfloat16),
                             format=plsc.PackFormat.INTERLEAVED)
        ci, di = plsc.unpack(plsc.bitcast(wi1, jnp.bfloat16),
                             format=plsc.PackFormat.INTERLEAVED)
        cj, dj = plsc.unpack(plsc.bitcast(wj1, jnp.bfloat16),
                             format=plsc.PackFormat.INTERLEAVED)
        return ai * aj + bi * bj + ci * cj + di * dj

    def unpack_idx(w):
        ii = lax.convert_element_type(w & jnp.uint32(0xFFFF), jnp.int32)
        jj = lax.convert_element_type(
            jnp.right_shift(w, jnp.uint32(16)), jnp.int32)
        return ii, jj

    def round_body(r, carry0):
        rbase = gbase + r * CHR

        def idx_src(sub):
            return idx_hbm.at[pl.ds(pl.multiple_of(rbase + sub * S, 8), S)]

        def compute_sub(sub, idxv):
            @plsc.parallel_loop(0, VPS, unroll=8)
            def vec_body(v):
                accw[pl.ds(v * 16, 16)] = plsc.bitcast(
                    dot4(*unpack_idx(idxv[pl.ds(v * 16, 16)])), jnp.uint32)
            pltpu.sync_copy(
                accw,
                shared.at[pl.ds(
                    pl.multiple_of(pg * 8 * CHR + q * CHR + sub * S, 8),
                    S)])

        pltpu.async_copy(idx_src(0), idxa, sema)

        def pair_body(k, carry_):
            sub0 = 2 * k
            sub1 = 2 * k + 1
            pltpu.make_async_copy(idx_src(sub0), idxa, sema).wait()
            pltpu.async_copy(idx_src(sub1), idxb, semb)
            compute_sub(sub0, idxa)
            pltpu.make_async_copy(idx_src(sub1), idxb, semb).wait()
            pltpu.async_copy(
                idx_src(jnp.minimum(sub1 + 1, NSUBR - 1)), idxa, sema)
            compute_sub(sub1, idxb)
            return carry_

        lax.fori_loop(0, NSUBR // 2, pair_body, 0)
        pltpu.make_async_copy(idx_src(NSUBR - 1), idxa, sema).wait()
        plsc.subcore_barrier()

        def wo_body(p, carry_):
            off = s * WR + p * PW
            goff = pl.multiple_of(
                (c * 8 + off // CHR) * CH + r * CHR + off % CHR, 8)
            pltpu.sync_copy(
                shared.at[pl.ds(pl.multiple_of(off, 8), PW)], idxa)
            pltpu.sync_copy(
                shared.at[pl.ds(pl.multiple_of(8 * CHR + off, 8), PW)], idxb)
            pltpu.sync_copy(carry_hbm.at[pl.ds(goff, PW)], wc)

            @plsc.parallel_loop(0, PW // 16, unroll=5)
            def add_body(t):
                sl = pl.ds(t * 16, 16)
                wc[sl] = (plsc.bitcast(idxa[sl], jnp.float32)
                          + plsc.bitcast(idxb[sl], jnp.float32) + wc[sl])
            pltpu.sync_copy(wc, out_hbm.at[pl.ds(goff, PW)])
            return carry_

        lax.fori_loop(0, NP, wo_body, 0)
        plsc.subcore_barrier()
        return carry0

    lax.fori_loop(0, R, round_body, 0)


def kernel(values, moment_idx, carry):
    vb = values.astype(jnp.bfloat16)
    u = lax.bitcast_convert_type(vb, jnp.uint16).astype(jnp.uint32)
    words = u[0::2] | (u[1::2] << jnp.uint32(16))
    tbl = lax.bitcast_convert_type(words, jnp.float32)

    mi = moment_idx.astype(jnp.uint32)
    idxp = mi[:, 0] | (mi[:, 1] << jnp.uint32(16))

    run = functools.partial(
        pl.kernel,
        out_type=jax.ShapeDtypeStruct((G,), jnp.float32),
        mesh=plsc.VectorSubcoreMesh(core_axis_name="c", subcore_axis_name="s"),
        compiler_params=pltpu.CompilerParams(needs_layout_passes=False),
        scratch_types=[
            pltpu.VMEM((N,), jnp.float32),
            pltpu.VMEM((N,), jnp.float32),
            pltpu.VMEM((S,), jnp.uint32),
            pltpu.VMEM((S,), jnp.uint32),
            pltpu.VMEM((S,), jnp.uint32),
            pltpu.VMEM((PW,), jnp.float32),
            pltpu.VMEM_SHARED((2 * 8 * CHR,), jnp.uint32),
            pltpu.SemaphoreType.DMA,
            pltpu.SemaphoreType.DMA,
        ],
    )(_body)
    return run(tbl, idxp, carry)
